# Initial kernel scaffold; baseline (speedup 1.0000x reference)
import jax
import jax.numpy as jnp
from jax.experimental import pallas as pl


def kernel(features, edge_index, edge_type, W1, q1, k1, W2, q2, k2, dec_w1, dec_b1, dec_w2, dec_b2):
    raise NotImplementedError("write your pallas kernel here")


# speedup vs baseline: 34.3888x; 34.3888x over previous
import functools

import jax
import jax.numpy as jnp
from jax import lax
from jax.experimental import pallas as pl
from jax.experimental.pallas import tpu as pltpu
from jax.experimental.pallas import tpu_sc as plsc

N = 10000
E = 320000
NW = 32
EPW = E // NW
B = 128
NCH = (EPW + B - 1) // B
EPAD = NCH * B
DEN = 10240

_SC_PARAMS = pltpu.CompilerParams(
    needs_layout_passes=False, use_tc_tiling_on_sc=False)
_MESH = plsc.VectorSubcoreMesh(core_axis_name="c", subcore_axis_name="s")


def _elu(x):
  return jnp.where(x > 0, x, jnp.exp(jnp.minimum(x, 0.0)) - 1.0)


def _scores1_tc(x, W1, q1, k1):
  blk = 1000

  def body(x_ref, w_ref, q_ref, k_ref, o_ref):
    qk = jnp.concatenate(
        [w_ref[0] @ q_ref[...], w_ref[1] @ q_ref[...],
         w_ref[0] @ k_ref[...], w_ref[1] @ k_ref[...]], axis=1)
    o_ref[...] = jnp.dot(x_ref[...], qk, preferred_element_type=jnp.float32)

  return pl.pallas_call(
      body,
      grid=(N // blk,),
      in_specs=[
          pl.BlockSpec((blk, 128), lambda i: (i, 0)),
          pl.BlockSpec((2, 128, 256), lambda i: (0, 0, 0)),
          pl.BlockSpec((256, 1), lambda i: (0, 0)),
          pl.BlockSpec((256, 1), lambda i: (0, 0)),
      ],
      out_specs=pl.BlockSpec((blk, 4), lambda i: (i, 0)),
      out_shape=jax.ShapeDtypeStruct((N, 4), jnp.float32),
  )(x, W1, q1, k1)


def _make_attn(flat_dst):
  @functools.partial(
      pl.kernel, mesh=_MESH, compiler_params=_SC_PARAMS,
      out_type=[jax.ShapeDtypeStruct((NW, EPAD), jnp.float32),
                jax.ShapeDtypeStruct((NW, EPAD), jnp.int32),
                jax.ShapeDtypeStruct((NW, NCH, B), jnp.int32),
                jax.ShapeDtypeStruct((2, DEN), jnp.float32)],
      scratch_types=[
          pltpu.VMEM((4 * N,), jnp.float32),
          pltpu.VMEM((EPW,), jnp.int32),
          pltpu.VMEM((EPW,), jnp.int32),
          pltpu.VMEM((EPW,), jnp.int32),
          pltpu.VMEM((EPAD,), jnp.float32),
          pltpu.VMEM((EPAD,), jnp.int32),
          pltpu.VMEM((NCH, B), jnp.int32),
          pltpu.VMEM((DEN,), jnp.float32),
          pltpu.VMEM((640,), jnp.float32),
          pltpu.VMEM((640,), jnp.float32),
          pltpu.VMEM_SHARED((16, DEN), jnp.float32),
          pltpu.SemaphoreType.DMA,
      ],
  )
  def attn(stab_hbm, src_hbm, dst_hbm, et_hbm,
           ex_hbm, fs_hbm, fd_hbm, denP_hbm,
           stab_v, src_v, dst_v, et_v, ex_v, fs_v, fd_v, den_v,
           dbuf_v, abuf_v, den_sh, sem):
    c = lax.axis_index("c")
    s = lax.axis_index("s")
    wid = c * 16 + s
    eoff = pl.multiple_of(wid * EPW, 8)

    pltpu.sync_copy(stab_hbm, stab_v)
    pltpu.sync_copy(src_hbm.at[pl.ds(eoff, EPW)], src_v)
    pltpu.sync_copy(dst_hbm.at[pl.ds(eoff, EPW)], dst_v)
    pltpu.sync_copy(et_hbm.at[pl.ds(eoff, EPW)], et_v)

    lane = lax.iota(jnp.int32, 16)
    qmask = (lane % 4) < 2
    big = jnp.float32(-3e38)

    def cbody(i, acc):
      for u in range(4):
        acc = jnp.maximum(acc, stab_v[pl.ds((i * 4 + u) * 16, 16)])
      return acc

    mx = lax.fori_loop(0, 4 * N // 64, cbody,
                       jnp.full((16,), big, jnp.float32))
    qmax = jnp.max(jnp.where(qmask, mx, big))
    kmax = jnp.max(jnp.where(qmask, big, mx))
    C = jnp.maximum(jnp.float32(0.0), qmax + kmax)

    def zden(i, _):
      den_v[pl.ds(i * 16, 16)] = jnp.zeros((16,), jnp.float32)
      return 0
    lax.fori_loop(0, DEN // 16, zden, 0)

    def abody(i, _):
      d16 = dst_v[pl.ds(i * 16, 16)]
      s16 = src_v[pl.ds(i * 16, 16)]
      t16 = et_v[pl.ds(i * 16, 16)]
      qi = plsc.load_gather(stab_v, [d16 * 4 + t16])
      kj = plsc.load_gather(stab_v, [s16 * 4 + 2 + t16])
      a = qi + kj
      a = jnp.where(a > 0, a, 0.2 * a)
      exv = jnp.exp(a - C)
      ex_v[pl.ds(i * 16, 16)] = exv
      plsc.addupdate_scatter(den_v, [d16], exv)
      if flat_dst:
        fs_v[pl.ds(i * 16, 16)] = s16
        fd_v[i // 8, pl.ds((i % 8) * 16, 16)] = t16 * N + d16
      else:
        fs_v[pl.ds(i * 16, 16)] = t16 * N + s16
        fd_v[i // 8, pl.ds((i % 8) * 16, 16)] = d16
      return 0
    lax.fori_loop(0, EPW // 16, abody, 0)
    for j in range((EPAD - EPW) // 16):
      ex_v[pl.ds(EPW + j * 16, 16)] = jnp.zeros((16,), jnp.float32)
      fs_v[pl.ds(EPW + j * 16, 16)] = jnp.zeros((16,), jnp.int32)
      fd_v[NCH - 1, pl.ds(B - (EPAD - EPW) + j * 16, 16)] = (
          jnp.zeros((16,), jnp.int32))

    pltpu.sync_copy(ex_v, ex_hbm.at[wid])
    pltpu.sync_copy(fs_v, fs_hbm.at[wid])
    pltpu.sync_copy(fd_v, fd_hbm.at[wid])

    pltpu.sync_copy(den_v, den_sh.at[s])
    plsc.subcore_barrier()
    doff = pl.multiple_of(s * 640, 8)

    def zabuf(i, _):
      abuf_v[pl.ds(i * 16, 16)] = jnp.zeros((16,), jnp.float32)
      return 0
    lax.fori_loop(0, 40, zabuf, 0)

    def dred(j, _):
      pltpu.sync_copy(den_sh.at[j, pl.ds(doff, 640)], dbuf_v)

      def dacc(k, _):
        abuf_v[pl.ds(k * 16, 16)] = (abuf_v[pl.ds(k * 16, 16)]
                                     + dbuf_v[pl.ds(k * 16, 16)])
        return 0
      lax.fori_loop(0, 40, dacc, 0)
      return 0
    lax.fori_loop(0, 16, dred, 0)
    pltpu.sync_copy(abuf_v, denP_hbm.at[c, pl.ds(doff, 640)])

  return attn


_attn1_sc = _make_attn(True)
_attn2_sc = _make_attn(False)


def _make_rows(ntab, nacc, dw):
  nvec = dw // 16
  tile_rows = nacc // 16
  ncopy = tile_rows // 125

  @functools.partial(
      pl.kernel, mesh=_MESH, compiler_params=_SC_PARAMS,
      out_type=jax.ShapeDtypeStruct((2, ntab, nacc, dw), jnp.float32),
      scratch_types=[
          pltpu.VMEM((EPAD,), jnp.float32),
          pltpu.VMEM((EPAD,), jnp.int32),
          pltpu.VMEM((NCH, B), jnp.int32),
          pltpu.VMEM((B, dw), jnp.float32),
          pltpu.VMEM_SHARED((nacc, dw), jnp.float32),
          pltpu.SemaphoreType.DMA,
      ],
  )
  def rows(*args):
    tab_hbms = args[:ntab]
    (ex_hbm, fs_hbm, fd_hbm, out_hbm,
     ex_v, fs_v, fd_v, rows_v, acc_sh, sem) = args[ntab:]
    c = lax.axis_index("c")
    s = lax.axis_index("s")
    wid = c * 16 + s

    pltpu.sync_copy(ex_hbm.at[wid], ex_v)
    pltpu.sync_copy(fs_hbm.at[wid], fs_v)
    pltpu.sync_copy(fd_hbm.at[wid], fd_v)

    def zrows(i, _):
      for u in range(nvec):
        rows_v[i, pl.ds(u * 16, 16)] = jnp.zeros((16,), jnp.float32)
      return 0
    lax.fori_loop(0, B, zrows, 0)
    for j in range(ncopy):
      pltpu.sync_copy(rows_v.at[pl.ds(0, 125), :],
                      acc_sh.at[pl.ds(s * tile_rows + j * 125, 125), :])
    plsc.subcore_barrier()

    for h in range(ntab):
      tab_hbm = tab_hbms[h]

      def rbody(j, _):
        off = pl.multiple_of(j * B, 8)
        pltpu.async_copy(tab_hbm.at[fs_v.at[pl.ds(off, B)]],
                         rows_v, sem).wait()

        def sbody(r, _):
          wv = plsc.load_gather(
              ex_v, [jnp.full((16,), j * B, jnp.int32) + r])
          for u in range(nvec):
            rows_v[r, pl.ds(u * 16, 16)] = rows_v[r, pl.ds(u * 16, 16)] * wv
          return 0
        lax.fori_loop(0, B, sbody, 0)
        pltpu.async_copy(rows_v, acc_sh.at[fd_v.at[j]], sem, add=True).wait()
        return 0
      lax.fori_loop(0, NCH, rbody, 0)

      plsc.subcore_barrier()
      for j in range(ncopy):
        roff = s * tile_rows + j * 125
        pltpu.sync_copy(acc_sh.at[pl.ds(roff, 125), :],
                        rows_v.at[pl.ds(0, 125), :])
        pltpu.sync_copy(rows_v.at[pl.ds(0, 125), :],
                        out_hbm.at[c, h, pl.ds(roff, 125), :])
      if h + 1 < ntab:
        def zrows2(i, _):
          for u in range(nvec):
            rows_v[i, pl.ds(u * 16, 16)] = jnp.zeros((16,), jnp.float32)
          return 0
        lax.fori_loop(0, 125, zrows2, 0)
        for j in range(ncopy):
          pltpu.sync_copy(rows_v.at[pl.ds(0, 125), :],
                          acc_sh.at[pl.ds(s * tile_rows + j * 125, 125), :])
        plsc.subcore_barrier()

  return rows


_rows1_sc = _make_rows(2, 2 * N, 64)
_rows2_sc = _make_rows(1, N, 32)


def _combine_tc(accP, denP, W1, W2, q2, k2):
  blk = 1000

  def body(a0_ref, a1_ref, den_ref, w1_ref, w2_ref, q2_ref, k2_ref,
           vtab_ref, stab_ref):
    acc0 = jnp.concatenate([a0_ref[0, 0] + a0_ref[1, 0],
                            a0_ref[0, 1] + a0_ref[1, 1]], axis=1)
    acc1 = jnp.concatenate([a1_ref[0, 0] + a1_ref[1, 0],
                            a1_ref[0, 1] + a1_ref[1, 1]], axis=1)
    den = den_ref[0, :, 0] + den_ref[1, :, 0]
    inv = (1.0 / (den + 1e-16))[:, None]
    out1 = (jnp.dot(acc0, w1_ref[0], preferred_element_type=jnp.float32)
            + jnp.dot(acc1, w1_ref[1], preferred_element_type=jnp.float32))
    h1 = _elu(out1 * inv)
    ones = jnp.ones((blk, 1), jnp.float32)
    zeros = jnp.zeros((blk, 1), jnp.float32)
    for r in range(2):
      v = jnp.dot(h1, w2_ref[r], preferred_element_type=jnp.float32)
      vtab_ref[r] = jnp.concatenate([v, ones, zeros], axis=1)
    qk2 = jnp.concatenate(
        [w2_ref[0] @ q2_ref[...], w2_ref[1] @ q2_ref[...],
         w2_ref[0] @ k2_ref[...], w2_ref[1] @ k2_ref[...]], axis=1)
    stab_ref[...] = jnp.dot(h1, qk2, preferred_element_type=jnp.float32)

  return pl.pallas_call(
      body,
      grid=(N // blk,),
      in_specs=[
          pl.BlockSpec((2, 2, blk, 64), lambda i: (0, 0, i, 0)),
          pl.BlockSpec((2, 2, blk, 64), lambda i: (0, 0, N // blk + i, 0)),
          pl.BlockSpec((2, blk, 1), lambda i: (0, i, 0)),
          pl.BlockSpec((2, 128, 256), lambda i: (0, 0, 0)),
          pl.BlockSpec((2, 256, 30), lambda i: (0, 0, 0)),
          pl.BlockSpec((30, 1), lambda i: (0, 0)),
          pl.BlockSpec((30, 1), lambda i: (0, 0)),
      ],
      out_specs=[
          pl.BlockSpec((2, blk, 32), lambda i: (0, i, 0)),
          pl.BlockSpec((blk, 4), lambda i: (i, 0)),
      ],
      out_shape=[jax.ShapeDtypeStruct((2, N, 32), jnp.float32),
                 jax.ShapeDtypeStruct((N, 4), jnp.float32)],
  )(accP, accP, denP.reshape(2, DEN, 1), W1, W2, q2, k2)


def _final_tc(acc2P, dec_w1, dec_b1, dec_w2, dec_b2):
  blk = 1000

  def body(a_ref, w1_ref, b1_ref, w2_ref, b2_ref, h2_ref, h3_ref):
    sacc = a_ref[0] + a_ref[1]
    h2 = _elu(sacc[:, :30] / (sacc[:, 30:31] + 1e-16))
    h2_ref[...] = h2
    t = jnp.dot(h2, w1_ref[...], preferred_element_type=jnp.float32) + b1_ref[...]
    h3_ref[...] = (jnp.dot(t, w2_ref[...], preferred_element_type=jnp.float32)
                   + b2_ref[...])

  return pl.pallas_call(
      body,
      grid=(N // blk,),
      in_specs=[
          pl.BlockSpec((2, blk, 32), lambda i: (0, i, 0)),
          pl.BlockSpec((30, 256), lambda i: (0, 0)),
          pl.BlockSpec((1, 256), lambda i: (0, 0)),
          pl.BlockSpec((256, 128), lambda i: (0, 0)),
          pl.BlockSpec((1, 128), lambda i: (0, 0)),
      ],
      out_specs=[
          pl.BlockSpec((blk, 30), lambda i: (i, 0)),
          pl.BlockSpec((blk, 128), lambda i: (i, 0)),
      ],
      out_shape=[jax.ShapeDtypeStruct((N, 30), jnp.float32),
                 jax.ShapeDtypeStruct((N, 128), jnp.float32)],
  )(acc2P, dec_w1, dec_b1, dec_w2, dec_b2)


def kernel(features, edge_index, edge_type, W1, q1, k1, W2, q2, k2,
           dec_w1, dec_b1, dec_w2, dec_b2):
  src = edge_index[0]
  dst = edge_index[1]
  et = edge_type
  xlo = features[:, :64]
  xhi = features[:, 64:]

  stab1 = _scores1_tc(features, W1, q1, k1)
  ex1, fs1, fd1, denP = _attn1_sc(stab1.reshape(-1), src, dst, et)
  accP = _rows1_sc(xlo, xhi, ex1, fs1, fd1)
  vtab2, stab2 = _combine_tc(accP, denP, W1, W2, q2, k2)
  ex2, fs2, fd2, _ = _attn2_sc(stab2.reshape(-1), src, dst, et)
  acc2P = _rows2_sc(vtab2.reshape(2 * N, 32), ex2, fs2, fd2)
  h2, h3 = _final_tc(acc2P.reshape(2, N, 32), dec_w1, dec_b1.reshape(1, -1),
                     dec_w2, dec_b2.reshape(1, -1))
  return (h2, h3)


# speedup vs baseline: 40.8719x; 1.1885x over previous
import functools

import jax
import jax.numpy as jnp
from jax import lax
from jax.experimental import pallas as pl
from jax.experimental.pallas import tpu as pltpu
from jax.experimental.pallas import tpu_sc as plsc

N = 10000
E = 320000
NW = 32
EPW = E // NW
B = 128
NCH = (EPW + B - 1) // B
EPAD = NCH * B
DEN = 10240

_SC_PARAMS = pltpu.CompilerParams(
    needs_layout_passes=False, use_tc_tiling_on_sc=False)
_MESH = plsc.VectorSubcoreMesh(core_axis_name="c", subcore_axis_name="s")


def _elu(x):
  return jnp.where(x > 0, x, jnp.exp(jnp.minimum(x, 0.0)) - 1.0)


def _scores1_tc(x, W1, q1, k1):
  blk = 1000

  def body(x_ref, w_ref, q_ref, k_ref, o_ref):
    qk = jnp.concatenate(
        [w_ref[0] @ q_ref[...], w_ref[1] @ q_ref[...],
         w_ref[0] @ k_ref[...], w_ref[1] @ k_ref[...]], axis=1)
    o_ref[...] = jnp.dot(x_ref[...], qk, preferred_element_type=jnp.float32)

  return pl.pallas_call(
      body,
      grid=(N // blk,),
      in_specs=[
          pl.BlockSpec((blk, 128), lambda i: (i, 0)),
          pl.BlockSpec((2, 128, 256), lambda i: (0, 0, 0)),
          pl.BlockSpec((256, 1), lambda i: (0, 0)),
          pl.BlockSpec((256, 1), lambda i: (0, 0)),
      ],
      out_specs=pl.BlockSpec((blk, 4), lambda i: (i, 0)),
      out_shape=jax.ShapeDtypeStruct((N, 4), jnp.float32),
  )(x, W1, q1, k1)


def _make_attn(flat_dst):
  @functools.partial(
      pl.kernel, mesh=_MESH, compiler_params=_SC_PARAMS,
      out_type=[jax.ShapeDtypeStruct((NW, EPAD), jnp.float32),
                jax.ShapeDtypeStruct((NW, EPAD), jnp.int32),
                jax.ShapeDtypeStruct((NW, NCH, B), jnp.int32),
                jax.ShapeDtypeStruct((2, DEN), jnp.float32)],
      scratch_types=[
          pltpu.VMEM((4 * N,), jnp.float32),
          pltpu.VMEM((EPW,), jnp.int32),
          pltpu.VMEM((EPW,), jnp.int32),
          pltpu.VMEM((EPW,), jnp.int32),
          pltpu.VMEM((EPAD,), jnp.float32),
          pltpu.VMEM((EPAD,), jnp.int32),
          pltpu.VMEM((NCH, B), jnp.int32),
          pltpu.VMEM((DEN,), jnp.float32),
          pltpu.VMEM((640,), jnp.float32),
          pltpu.VMEM((640,), jnp.float32),
          pltpu.VMEM_SHARED((16, DEN), jnp.float32),
          pltpu.SemaphoreType.DMA,
      ],
  )
  def attn(stab_hbm, src_hbm, dst_hbm, et_hbm,
           ex_hbm, fs_hbm, fd_hbm, denP_hbm,
           stab_v, src_v, dst_v, et_v, ex_v, fs_v, fd_v, den_v,
           dbuf_v, abuf_v, den_sh, sem):
    c = lax.axis_index("c")
    s = lax.axis_index("s")
    wid = c * 16 + s
    eoff = pl.multiple_of(wid * EPW, 8)

    pltpu.sync_copy(stab_hbm, stab_v)
    pltpu.sync_copy(src_hbm.at[pl.ds(eoff, EPW)], src_v)
    pltpu.sync_copy(dst_hbm.at[pl.ds(eoff, EPW)], dst_v)
    pltpu.sync_copy(et_hbm.at[pl.ds(eoff, EPW)], et_v)

    lane = lax.iota(jnp.int32, 16)
    qmask = (lane % 4) < 2
    big = jnp.float32(-3e38)

    def cbody(i, acc):
      for u in range(4):
        acc = jnp.maximum(acc, stab_v[pl.ds((i * 4 + u) * 16, 16)])
      return acc

    mx = lax.fori_loop(0, 4 * N // 64, cbody,
                       jnp.full((16,), big, jnp.float32))
    qmax = jnp.max(jnp.where(qmask, mx, big))
    kmax = jnp.max(jnp.where(qmask, big, mx))
    C = jnp.maximum(jnp.float32(0.0), qmax + kmax)

    def zden(i, _):
      den_v[pl.ds(i * 16, 16)] = jnp.zeros((16,), jnp.float32)
      return 0
    lax.fori_loop(0, DEN // 16, zden, 0)

    def abody(i, _):
      d16 = dst_v[pl.ds(i * 16, 16)]
      s16 = src_v[pl.ds(i * 16, 16)]
      t16 = et_v[pl.ds(i * 16, 16)]
      qi = plsc.load_gather(stab_v, [d16 * 4 + t16])
      kj = plsc.load_gather(stab_v, [s16 * 4 + 2 + t16])
      a = qi + kj
      a = jnp.where(a > 0, a, 0.2 * a)
      exv = jnp.exp(a - C)
      ex_v[pl.ds(i * 16, 16)] = exv
      plsc.addupdate_scatter(den_v, [d16], exv)
      if flat_dst:
        fs_v[pl.ds(i * 16, 16)] = s16
        fd_v[i // 8, pl.ds((i % 8) * 16, 16)] = t16 * N + d16
      else:
        fs_v[pl.ds(i * 16, 16)] = t16 * N + s16
        fd_v[i // 8, pl.ds((i % 8) * 16, 16)] = d16
      return 0
    lax.fori_loop(0, EPW // 16, abody, 0)
    for j in range((EPAD - EPW) // 16):
      ex_v[pl.ds(EPW + j * 16, 16)] = jnp.zeros((16,), jnp.float32)
      fs_v[pl.ds(EPW + j * 16, 16)] = jnp.zeros((16,), jnp.int32)
      fd_v[NCH - 1, pl.ds(B - (EPAD - EPW) + j * 16, 16)] = (
          jnp.zeros((16,), jnp.int32))

    pltpu.sync_copy(ex_v, ex_hbm.at[wid])
    pltpu.sync_copy(fs_v, fs_hbm.at[wid])
    pltpu.sync_copy(fd_v, fd_hbm.at[wid])

    pltpu.sync_copy(den_v, den_sh.at[s])
    plsc.subcore_barrier()
    doff = pl.multiple_of(s * 640, 8)

    def zabuf(i, _):
      abuf_v[pl.ds(i * 16, 16)] = jnp.zeros((16,), jnp.float32)
      return 0
    lax.fori_loop(0, 40, zabuf, 0)

    def dred(j, _):
      pltpu.sync_copy(den_sh.at[j, pl.ds(doff, 640)], dbuf_v)

      def dacc(k, _):
        abuf_v[pl.ds(k * 16, 16)] = (abuf_v[pl.ds(k * 16, 16)]
                                     + dbuf_v[pl.ds(k * 16, 16)])
        return 0
      lax.fori_loop(0, 40, dacc, 0)
      return 0
    lax.fori_loop(0, 16, dred, 0)
    pltpu.sync_copy(abuf_v, denP_hbm.at[c, pl.ds(doff, 640)])

  return attn


_attn1_sc = _make_attn(True)
_attn2_sc = _make_attn(False)


def _make_rows(ntab, nacc, dw):
  nvec = dw // 16
  tile_rows = nacc // 16
  ncopy = tile_rows // 125

  @functools.partial(
      pl.kernel, mesh=_MESH, compiler_params=_SC_PARAMS,
      out_type=jax.ShapeDtypeStruct((2, ntab, nacc, dw), jnp.float32),
      scratch_types=[
          pltpu.VMEM((EPAD,), jnp.float32),
          pltpu.VMEM((EPAD,), jnp.int32),
          pltpu.VMEM((NCH, B), jnp.int32),
          pltpu.VMEM((B, dw), jnp.float32),
          pltpu.VMEM((B, dw), jnp.float32),
          pltpu.VMEM_SHARED((nacc, dw), jnp.float32),
          pltpu.SemaphoreType.DMA,
          pltpu.SemaphoreType.DMA,
          pltpu.SemaphoreType.DMA,
          pltpu.SemaphoreType.DMA,
      ],
  )
  def rows(*args):
    tab_hbms = args[:ntab]
    (ex_hbm, fs_hbm, fd_hbm, out_hbm,
     ex_v, fs_v, fd_v, buf0_v, buf1_v, acc_sh, g0, g1, s0, s1) = args[ntab:]
    c = lax.axis_index("c")
    s = lax.axis_index("s")
    wid = c * 16 + s

    pltpu.sync_copy(ex_hbm.at[wid], ex_v)
    pltpu.sync_copy(fs_hbm.at[wid], fs_v)
    pltpu.sync_copy(fd_hbm.at[wid], fd_v)

    def zbuf(buf, nrows):
      def zr(i, _):
        for u in range(nvec):
          buf[i, pl.ds(u * 16, 16)] = jnp.zeros((16,), jnp.float32)
        return 0
      lax.fori_loop(0, nrows, zr, 0)

    def zero_acc_slice():
      for j in range(ncopy):
        pltpu.sync_copy(buf0_v.at[pl.ds(0, 125), :],
                        acc_sh.at[pl.ds(s * tile_rows + j * 125, 125), :])

    def scale(buf, jchunk):
      base = jchunk * B

      def s4(r4, _):
        for rr in range(4):
          r = r4 * 4 + rr
          wv = plsc.load_gather(ex_v, [jnp.full((16,), base, jnp.int32) + r])
          for u in range(nvec):
            buf[r, pl.ds(u * 16, 16)] = buf[r, pl.ds(u * 16, 16)] * wv
        return 0
      lax.fori_loop(0, B // 4, s4, 0)

    def gather(tab_hbm, jchunk, buf, sem):
      off = pl.multiple_of(jchunk * B, 8)
      return pltpu.async_copy(tab_hbm.at[fs_v.at[pl.ds(off, B)]], buf, sem)

    def scatter(jchunk, buf, sem):
      return pltpu.async_copy(buf, acc_sh.at[fd_v.at[jchunk]], sem, add=True)

    zbuf(buf0_v, B)
    zero_acc_slice()
    plsc.subcore_barrier()

    for h in range(ntab):
      tab_hbm = tab_hbms[h]

      def pbody(p, _):
        j0 = p * 2
        j1 = j0 + 1
        cg0 = gather(tab_hbm, j0, buf0_v, g0)
        cg1 = gather(tab_hbm, j1, buf1_v, g1)
        cg0.wait()
        scale(buf0_v, j0)
        cs0 = scatter(j0, buf0_v, s0)
        cg1.wait()
        scale(buf1_v, j1)
        cs1 = scatter(j1, buf1_v, s1)
        cs0.wait()
        cs1.wait()
        return 0
      lax.fori_loop(0, NCH // 2, pbody, 0)
      if NCH % 2:
        jlast = NCH - 1
        cg = gather(tab_hbm, jlast, buf0_v, g0)
        cg.wait()
        scale(buf0_v, jlast)
        scatter(jlast, buf0_v, s0).wait()

      plsc.subcore_barrier()
      handles = []
      for j in range(ncopy):
        buf = buf0_v if j % 2 == 0 else buf1_v
        sem = s0 if j % 2 == 0 else s1
        if j >= 2:
          handles[j - 2].wait()
        roff = s * tile_rows + j * 125
        pltpu.sync_copy(acc_sh.at[pl.ds(roff, 125), :], buf.at[pl.ds(0, 125), :])
        handles.append(
            pltpu.async_copy(buf.at[pl.ds(0, 125), :],
                             out_hbm.at[c, h, pl.ds(roff, 125), :], sem))
      for hd in handles[-2:]:
        hd.wait()
      if h + 1 < ntab:
        zbuf(buf0_v, 125)
        zero_acc_slice()
        plsc.subcore_barrier()

  return rows


_rows1_sc = _make_rows(2, 2 * N, 64)
_rows2_sc = _make_rows(1, N, 32)


def _combine_tc(accP, denP, W1, W2, q2, k2):
  blk = 1000

  def body(a0_ref, a1_ref, den_ref, w1_ref, w2_ref, q2_ref, k2_ref,
           vtab_ref, stab_ref):
    acc0 = jnp.concatenate([a0_ref[0, 0] + a0_ref[1, 0],
                            a0_ref[0, 1] + a0_ref[1, 1]], axis=1)
    acc1 = jnp.concatenate([a1_ref[0, 0] + a1_ref[1, 0],
                            a1_ref[0, 1] + a1_ref[1, 1]], axis=1)
    den = den_ref[0, :, 0] + den_ref[1, :, 0]
    inv = (1.0 / (den + 1e-16))[:, None]
    out1 = (jnp.dot(acc0, w1_ref[0], preferred_element_type=jnp.float32)
            + jnp.dot(acc1, w1_ref[1], preferred_element_type=jnp.float32))
    h1 = _elu(out1 * inv)
    ones = jnp.ones((blk, 1), jnp.float32)
    zeros = jnp.zeros((blk, 1), jnp.float32)
    for r in range(2):
      v = jnp.dot(h1, w2_ref[r], preferred_element_type=jnp.float32)
      vtab_ref[r] = jnp.concatenate([v, ones, zeros], axis=1)
    qk2 = jnp.concatenate(
        [w2_ref[0] @ q2_ref[...], w2_ref[1] @ q2_ref[...],
         w2_ref[0] @ k2_ref[...], w2_ref[1] @ k2_ref[...]], axis=1)
    stab_ref[...] = jnp.dot(h1, qk2, preferred_element_type=jnp.float32)

  return pl.pallas_call(
      body,
      grid=(N // blk,),
      in_specs=[
          pl.BlockSpec((2, 2, blk, 64), lambda i: (0, 0, i, 0)),
          pl.BlockSpec((2, 2, blk, 64), lambda i: (0, 0, N // blk + i, 0)),
          pl.BlockSpec((2, blk, 1), lambda i: (0, i, 0)),
          pl.BlockSpec((2, 128, 256), lambda i: (0, 0, 0)),
          pl.BlockSpec((2, 256, 30), lambda i: (0, 0, 0)),
          pl.BlockSpec((30, 1), lambda i: (0, 0)),
          pl.BlockSpec((30, 1), lambda i: (0, 0)),
      ],
      out_specs=[
          pl.BlockSpec((2, blk, 32), lambda i: (0, i, 0)),
          pl.BlockSpec((blk, 4), lambda i: (i, 0)),
      ],
      out_shape=[jax.ShapeDtypeStruct((2, N, 32), jnp.float32),
                 jax.ShapeDtypeStruct((N, 4), jnp.float32)],
  )(accP, accP, denP.reshape(2, DEN, 1), W1, W2, q2, k2)


def _final_tc(acc2P, dec_w1, dec_b1, dec_w2, dec_b2):
  blk = 1000

  def body(a_ref, w1_ref, b1_ref, w2_ref, b2_ref, h2_ref, h3_ref):
    sacc = a_ref[0] + a_ref[1]
    h2 = _elu(sacc[:, :30] / (sacc[:, 30:31] + 1e-16))
    h2_ref[...] = h2
    t = jnp.dot(h2, w1_ref[...], preferred_element_type=jnp.float32) + b1_ref[...]
    h3_ref[...] = (jnp.dot(t, w2_ref[...], preferred_element_type=jnp.float32)
                   + b2_ref[...])

  return pl.pallas_call(
      body,
      grid=(N // blk,),
      in_specs=[
          pl.BlockSpec((2, blk, 32), lambda i: (0, i, 0)),
          pl.BlockSpec((30, 256), lambda i: (0, 0)),
          pl.BlockSpec((1, 256), lambda i: (0, 0)),
          pl.BlockSpec((256, 128), lambda i: (0, 0)),
          pl.BlockSpec((1, 128), lambda i: (0, 0)),
      ],
      out_specs=[
          pl.BlockSpec((blk, 30), lambda i: (i, 0)),
          pl.BlockSpec((blk, 128), lambda i: (i, 0)),
      ],
      out_shape=[jax.ShapeDtypeStruct((N, 30), jnp.float32),
                 jax.ShapeDtypeStruct((N, 128), jnp.float32)],
  )(acc2P, dec_w1, dec_b1, dec_w2, dec_b2)


def kernel(features, edge_index, edge_type, W1, q1, k1, W2, q2, k2,
           dec_w1, dec_b1, dec_w2, dec_b2):
  src = edge_index[0]
  dst = edge_index[1]
  et = edge_type
  xlo = features[:, :64]
  xhi = features[:, 64:]

  stab1 = _scores1_tc(features, W1, q1, k1)
  ex1, fs1, fd1, denP = _attn1_sc(stab1.reshape(-1), src, dst, et)
  accP = _rows1_sc(xlo, xhi, ex1, fs1, fd1)
  vtab2, stab2 = _combine_tc(accP, denP, W1, W2, q2, k2)
  ex2, fs2, fd2, _ = _attn2_sc(stab2.reshape(-1), src, dst, et)
  acc2P = _rows2_sc(vtab2.reshape(2 * N, 32), ex2, fs2, fd2)
  h2, h3 = _final_tc(acc2P.reshape(2, N, 32), dec_w1, dec_b1.reshape(1, -1),
                     dec_w2, dec_b2.reshape(1, -1))
  return (h2, h3)
